# baseline (speedup 1.0000x reference)
import functools

import jax
import jax.numpy as jnp
from jax import lax
from jax.experimental import pallas as pl
from jax.experimental.pallas import tpu as pltpu
from jax.experimental.pallas import tpu_sc as plsc

NC = 2
NS = 16
NW = NC * NS

N_NODES = 100000
MEM_DIM = 500
B = 16384
BPW = B // NW
GCH = 64
NCHUNK = BPW // GCH

_mesh = plsc.VectorSubcoreMesh(core_axis_name="c", subcore_axis_name="s",
                               num_cores=NC, num_subcores=NS)


def _gather_body(ids_hbm, mem_hbm, lu_hbm, h_out, lu_out,
                 idx_v, hbuf, lubuf, sem1, sem2):
    c = lax.axis_index("c")
    s = lax.axis_index("s")
    wid = s * NC + c
    base = wid * BPW
    pltpu.sync_copy(ids_hbm.at[pl.ds(wid * NCHUNK, NCHUNK)], idx_v)
    for k in range(NCHUNK):
        row = idx_v.at[k]
        cp2 = pltpu.async_copy(lu_hbm.at[row], lubuf, sem2)
        for q in range(GCH // 16):
            v = idx_v[k, pl.ds(q * 16, 16)]
            for j in range(16):
                pltpu.make_async_copy(
                    mem_hbm.at[pl.ds(v[j], 1)],
                    hbuf.at[pl.ds(q * 16 + j, 1)],
                    sem1,
                ).start()
        pltpu.make_async_copy(mem_hbm.at[pl.ds(0, GCH)], hbuf, sem1).wait()
        pltpu.sync_copy(hbuf, h_out.at[pl.ds(base + k * GCH, GCH)])
        cp2.wait()
        pltpu.sync_copy(lubuf, lu_out.at[pl.ds(base + k * GCH, GCH)])


_gather_call = pl.kernel(
    _gather_body,
    out_type=[
        jax.ShapeDtypeStruct((B, MEM_DIM), jnp.float32),
        jax.ShapeDtypeStruct((B,), jnp.float32),
    ],
    mesh=_mesh,
    scratch_types=[
        pltpu.VMEM((NCHUNK, GCH), jnp.int32),
        pltpu.VMEM((GCH, MEM_DIM), jnp.float32),
        pltpu.VMEM((GCH,), jnp.float32),
        pltpu.SemaphoreType.DMA,
        pltpu.SemaphoreType.DMA,
    ],
    compiler_params=pltpu.CompilerParams(use_tc_tiling_on_sc=False),
)


BB = 512
NBB = B // BB


def _tc_body(h, ef, et, lu, wt, bt, w1a, w1b, w1c, b1, w2, b2,
             wxr, wxz, wxn, whr, whz, whn, bxr, bxz, bxn, bhr, bhz, bhn,
             hn_out):
    f32 = jnp.float32
    hh = h[...]
    td = et[...] - lu[...]
    te = jnp.cos(td * wt[...] + bt[...])
    hid = (jnp.dot(hh, w1a[...], preferred_element_type=f32)
           + jnp.dot(ef[...], w1b[...], preferred_element_type=f32)
           + jnp.dot(te, w1c[...], preferred_element_type=f32)
           + b1[...])
    hid = jnp.maximum(hid, 0.0)
    msg = jnp.dot(hid, w2[...], preferred_element_type=f32) + b2[...]
    xr = jnp.dot(msg, wxr[...], preferred_element_type=f32) + bxr[...]
    xz = jnp.dot(msg, wxz[...], preferred_element_type=f32) + bxz[...]
    xn = jnp.dot(msg, wxn[...], preferred_element_type=f32) + bxn[...]
    hr = jnp.dot(hh, whr[...], preferred_element_type=f32) + bhr[...]
    hz = jnp.dot(hh, whz[...], preferred_element_type=f32) + bhz[...]
    hn = jnp.dot(hh, whn[...], preferred_element_type=f32) + bhn[...]
    r = jax.nn.sigmoid(xr + hr)
    z = jax.nn.sigmoid(xz + hz)
    n = jnp.tanh(xn + r * hn)
    hn_out[...] = (1.0 - z) * n + z * hh


def _const_spec(shape):
    nd = len(shape)
    return pl.BlockSpec(shape, lambda i: (0,) * nd)


def _tc_compute(h, ef, et1, lu1, wt2, bt2, w1a, w1b, w1c, b1_2, w2, b2_2,
                gw, gb):
    in_specs = [
        pl.BlockSpec((BB, MEM_DIM), lambda i: (i, 0)),
        pl.BlockSpec((BB, ef.shape[1]), lambda i: (i, 0)),
        pl.BlockSpec((BB, 1), lambda i: (i, 0)),
        pl.BlockSpec((BB, 1), lambda i: (i, 0)),
        _const_spec(wt2.shape),
        _const_spec(bt2.shape),
        _const_spec(w1a.shape),
        _const_spec(w1b.shape),
        _const_spec(w1c.shape),
        _const_spec(b1_2.shape),
        _const_spec(w2.shape),
        _const_spec(b2_2.shape),
    ] + [_const_spec(w.shape) for w in gw] + [_const_spec(b.shape) for b in gb]
    return pl.pallas_call(
        _tc_body,
        grid=(NBB,),
        in_specs=in_specs,
        out_specs=pl.BlockSpec((BB, MEM_DIM), lambda i: (i, 0)),
        out_shape=jax.ShapeDtypeStruct((B, MEM_DIM), jnp.float32),
    )(h, ef, et1, lu1, wt2, bt2, w1a, w1b, w1c, b1_2, w2, b2_2, *gw, *gb)


def kernel(memory, last_update, node_ids, edge_feats, edge_times,
           w_t, b_t, W1, b1, W2, b2, W_ih, b_ih, W_hh, b_hh):
    ids2d = node_ids.astype(jnp.int32).reshape(B // GCH, GCH)
    h, lu = _gather_call(ids2d, memory, last_update)

    w1a = W1[:MEM_DIM]
    w1b = W1[MEM_DIM:MEM_DIM + 17]
    w1c = W1[MEM_DIM + 17:]
    gw = [W_ih[:, :MEM_DIM], W_ih[:, MEM_DIM:2 * MEM_DIM], W_ih[:, 2 * MEM_DIM:],
          W_hh[:, :MEM_DIM], W_hh[:, MEM_DIM:2 * MEM_DIM], W_hh[:, 2 * MEM_DIM:]]
    gb = [b_ih[:MEM_DIM][None], b_ih[MEM_DIM:2 * MEM_DIM][None], b_ih[2 * MEM_DIM:][None],
          b_hh[:MEM_DIM][None], b_hh[MEM_DIM:2 * MEM_DIM][None], b_hh[2 * MEM_DIM:][None]]
    h_new = _tc_compute(h, edge_feats, edge_times[:, None], lu[:, None],
                        w_t[None], b_t[None], w1a, w1b, w1c, b1[None],
                        W2, b2[None], gw, gb)

    return memory.at[node_ids].set(h_new)
